# baseline (speedup 1.0000x reference)
import functools
import math

import jax
import jax.numpy as jnp
from jax import lax
from jax.experimental import pallas as pl
from jax.experimental.pallas import tpu as pltpu
from jax.experimental.pallas import tpu_sc as plsc

ALPHA = 0.1
B, M, C = 1024, 128, 384
NC, NS = 2, 16
NW = NC * NS
BPW = B // NW


def _compute_body(x_ref, mm_ref, z_ref, kl_ref, idx_ref, rows_ref):
    X = x_ref[...]
    MM = mm_ref[...]
    sims = jax.lax.dot_general(X, MM, (((1,), (1,)), ((), ())),
                               preferred_element_type=jnp.float32)
    closest = jnp.argmax(sims, axis=1)
    onehot = (jax.lax.broadcasted_iota(jnp.int32, (B, M), 1)
              == closest[:, None])
    oh_f = onehot.astype(jnp.float32)
    gathered = jax.lax.dot_general(oh_f, MM, (((1,), (0,)), ((), ())),
                                   preferred_element_type=jnp.float32)
    diff = X - gathered
    delta = ALPHA * diff
    xsq = jnp.sum(X * X, axis=1)
    s_at = jnp.sum(sims * oh_f, axis=1)
    corr = (1.0 - ALPHA) * s_at + ALPHA * xsq
    scores = jnp.where(onehot, corr[:, None], sims) * (1.0 / math.sqrt(C))
    smax = jnp.max(scores, axis=1, keepdims=True)
    e = jnp.exp(scores - smax)
    w = e / jnp.sum(e, axis=1, keepdims=True)
    z = jax.lax.dot_general(w, MM, (((1,), (0,)), ((), ())),
                            preferred_element_type=jnp.float32)
    w_at = jnp.sum(w * oh_f, axis=1)
    z = z + w_at[:, None] * delta
    z_ref[...] = z
    kl_ref[...] = 0.5 * (jnp.sum(diff * diff, axis=1)
                         + jnp.sum((z - X) ** 2, axis=1))
    idx_ref[...] = closest + M * jax.lax.broadcasted_iota(jnp.int32, (B,), 0)
    rows_ref[...] = gathered + delta


def _sc_broadcast_body(mm_hbm, post_hbm, mm2_v, bsem):
    wid = lax.axis_index("s") * NC + lax.axis_index("c")
    base = wid * BPW
    pltpu.sync_copy(mm_hbm, mm2_v.at[pl.ds(0, M)])
    pltpu.sync_copy(mm_hbm, mm2_v.at[pl.ds(M, M)])
    for b in range(0, BPW, 2):
        pltpu.async_copy(mm2_v, post_hbm.at[pl.ds((base + b) * M, 2 * M)], bsem)
    for b in range(0, BPW, 2):
        pltpu.make_async_copy(mm2_v, post_hbm.at[pl.ds((base + b) * M, 2 * M)],
                              bsem).wait()


_sc_broadcast = functools.partial(
    pl.kernel,
    out_type=jax.ShapeDtypeStruct((B * M, C), jnp.float32),
    mesh=plsc.VectorSubcoreMesh(core_axis_name="c", subcore_axis_name="s"),
    scratch_types=[
        pltpu.VMEM((2 * M, C), jnp.float32),
        pltpu.SemaphoreType.DMA,
    ],
)(_sc_broadcast_body)


def _sc_scatter_body(rows_hbm, idx_hbm, post_ref, rows_v, idx_v, ssem):
    wid = lax.axis_index("s") * NC + lax.axis_index("c")
    base = wid * BPW
    pltpu.sync_copy(rows_hbm.at[pl.ds(base, BPW)], rows_v)
    pltpu.sync_copy(idx_hbm.at[pl.ds(base, BPW)], idx_v)
    pltpu.async_copy(rows_v, post_ref.at[idx_v], ssem).wait()


_sc_scatter = functools.partial(
    pl.kernel,
    out_type=(),
    mesh=plsc.VectorSubcoreMesh(core_axis_name="c", subcore_axis_name="s"),
    scratch_types=[
        pltpu.VMEM((BPW, C), jnp.float32),
        pltpu.VMEM((BPW,), jnp.int32),
        pltpu.SemaphoreType.DMA,
    ],
)(_sc_scatter_body)


def kernel(input_encoded, memory_mean, memory_logvar):
    del memory_logvar

    post_flat = _sc_broadcast(memory_mean)

    z, kl, flat_idx, new_rows = pl.pallas_call(
        _compute_body,
        out_shape=[
            jax.ShapeDtypeStruct((B, C), jnp.float32),
            jax.ShapeDtypeStruct((B,), jnp.float32),
            jax.ShapeDtypeStruct((B,), jnp.int32),
            jax.ShapeDtypeStruct((B, C), jnp.float32),
        ],
    )(input_encoded, memory_mean)

    post_ref = jax.new_ref(post_flat)
    _sc_scatter(new_rows, flat_idx, post_ref)
    posterior = post_ref[...].reshape(B, M, C)

    return z, posterior, kl


# speedup vs baseline: 1.0689x; 1.0689x over previous
import functools
import math

import jax
import jax.numpy as jnp
from jax import lax
from jax.experimental import pallas as pl
from jax.experimental.pallas import tpu as pltpu
from jax.experimental.pallas import tpu_sc as plsc

ALPHA = 0.1
B, M, C = 1024, 128, 384
NC, NS = 2, 16
NW = NC * NS
BPW = B // NW


def _compute_body(x_ref, mm_ref, z_ref, kl_ref, idx_ref, rows_ref):
    X = x_ref[...]
    MM = mm_ref[...]
    sims = jax.lax.dot_general(X, MM, (((1,), (1,)), ((), ())),
                               preferred_element_type=jnp.float32)
    closest = jnp.argmax(sims, axis=1)
    onehot = (jax.lax.broadcasted_iota(jnp.int32, (B, M), 1)
              == closest[:, None])
    oh_f = onehot.astype(jnp.float32)
    gathered = jax.lax.dot_general(oh_f, MM, (((1,), (0,)), ((), ())),
                                   preferred_element_type=jnp.float32)
    diff = X - gathered
    delta = ALPHA * diff
    xsq = jnp.sum(X * X, axis=1)
    s_at = jnp.sum(sims * oh_f, axis=1)
    corr = (1.0 - ALPHA) * s_at + ALPHA * xsq
    scores = jnp.where(onehot, corr[:, None], sims) * (1.0 / math.sqrt(C))
    smax = jnp.max(scores, axis=1, keepdims=True)
    e = jnp.exp(scores - smax)
    w = e / jnp.sum(e, axis=1, keepdims=True)
    z = jax.lax.dot_general(w, MM, (((1,), (0,)), ((), ())),
                            preferred_element_type=jnp.float32)
    w_at = jnp.sum(w * oh_f, axis=1)
    z = z + w_at[:, None] * delta
    z_ref[...] = z
    kl_ref[...] = 0.5 * (jnp.sum(diff * diff, axis=1)
                         + jnp.sum((z - X) ** 2, axis=1))
    idx_ref[...] = closest + M * jax.lax.broadcasted_iota(jnp.int32, (B,), 0)
    rows_ref[...] = gathered + delta


def _sc_broadcast_body(mm_hbm, post_hbm, mm_v, bsem):
    wid = lax.axis_index("s") * NC + lax.axis_index("c")
    base = wid * BPW
    pltpu.sync_copy(mm_hbm, mm_v)
    for b in range(BPW):
        pltpu.async_copy(mm_v, post_hbm.at[pl.ds((base + b) * M, M)], bsem)
    for b in range(BPW):
        pltpu.make_async_copy(mm_v, post_hbm.at[pl.ds((base + b) * M, M)],
                              bsem).wait()


_sc_broadcast = functools.partial(
    pl.kernel,
    out_type=jax.ShapeDtypeStruct((B * M, C), jnp.float32),
    mesh=plsc.VectorSubcoreMesh(core_axis_name="c", subcore_axis_name="s"),
    scratch_types=[
        pltpu.VMEM((M, C), jnp.float32),
        pltpu.SemaphoreType.DMA,
    ],
)(_sc_broadcast_body)


def _sc_scatter_body(rows_hbm, idx_hbm, post_ref, rows_v, idx_v, ssem):
    wid = lax.axis_index("s") * NC + lax.axis_index("c")
    base = wid * BPW
    pltpu.sync_copy(rows_hbm.at[pl.ds(base, BPW)], rows_v)
    pltpu.sync_copy(idx_hbm.at[pl.ds(base, BPW)], idx_v)
    pltpu.async_copy(rows_v, post_ref.at[idx_v], ssem).wait()


_sc_scatter = functools.partial(
    pl.kernel,
    out_type=(),
    mesh=plsc.VectorSubcoreMesh(core_axis_name="c", subcore_axis_name="s"),
    scratch_types=[
        pltpu.VMEM((BPW, C), jnp.float32),
        pltpu.VMEM((BPW,), jnp.int32),
        pltpu.SemaphoreType.DMA,
    ],
)(_sc_scatter_body)


def kernel(input_encoded, memory_mean, memory_logvar):
    del memory_logvar

    post_flat = _sc_broadcast(memory_mean)

    z, kl, flat_idx, new_rows = pl.pallas_call(
        _compute_body,
        out_shape=[
            jax.ShapeDtypeStruct((B, C), jnp.float32),
            jax.ShapeDtypeStruct((B,), jnp.float32),
            jax.ShapeDtypeStruct((B,), jnp.int32),
            jax.ShapeDtypeStruct((B, C), jnp.float32),
        ],
    )(input_encoded, memory_mean)

    post_ref = jax.new_ref(post_flat)
    _sc_scatter(new_rows, flat_idx, post_ref)
    posterior = post_ref[...].reshape(B, M, C)

    return z, posterior, kl


# speedup vs baseline: 1.1244x; 1.0519x over previous
import functools
import math

import jax
import jax.numpy as jnp
from jax import lax
from jax.experimental import pallas as pl
from jax.experimental.pallas import tpu as pltpu
from jax.experimental.pallas import tpu_sc as plsc

ALPHA = 0.1
B, M, C = 1024, 128, 384
NC, NS = 2, 16
NW = NC * NS
BPW = B // NW


def _compute_body(x_ref, mm_ref, z_ref, kl_ref, idx_ref, rows_ref):
    X = x_ref[...]
    MM = mm_ref[...]
    sims = jax.lax.dot_general(X, MM, (((1,), (1,)), ((), ())),
                               preferred_element_type=jnp.float32)
    closest = jnp.argmax(sims, axis=1)
    onehot = (jax.lax.broadcasted_iota(jnp.int32, (B, M), 1)
              == closest[:, None])
    oh_f = onehot.astype(jnp.float32)
    gathered = jax.lax.dot_general(oh_f, MM, (((1,), (0,)), ((), ())),
                                   preferred_element_type=jnp.float32)
    diff = X - gathered
    delta = ALPHA * diff
    xsq = jnp.sum(X * X, axis=1)
    s_at = jnp.sum(sims * oh_f, axis=1)
    corr = (1.0 - ALPHA) * s_at + ALPHA * xsq
    scores = jnp.where(onehot, corr[:, None], sims) * (1.0 / math.sqrt(C))
    smax = jnp.max(scores, axis=1, keepdims=True)
    e = jnp.exp(scores - smax)
    w = e / jnp.sum(e, axis=1, keepdims=True)
    z = jax.lax.dot_general(w, MM, (((1,), (0,)), ((), ())),
                            preferred_element_type=jnp.float32)
    w_at = jnp.sum(w * oh_f, axis=1)
    z = z + w_at[:, None] * delta
    z_ref[...] = z
    kl_ref[...] = 0.5 * (jnp.sum(diff * diff, axis=1)
                         + jnp.sum((z - X) ** 2, axis=1))
    idx_ref[...] = closest + M * jax.lax.broadcasted_iota(jnp.int32, (B,), 0)
    rows_ref[...] = gathered + delta


def _sc_writer_body(mm_hbm, rows_hbm, idx_hbm, post_hbm,
                    mm_v, rows_v, idx_v, bsem, ssem):
    wid = lax.axis_index("s") * NC + lax.axis_index("c")
    base = wid * BPW
    pltpu.sync_copy(mm_hbm, mm_v)
    pltpu.sync_copy(rows_hbm.at[pl.ds(base, BPW)], rows_v)
    pltpu.sync_copy(idx_hbm.at[pl.ds(base, BPW)], idx_v)

    def fire(b, _):
        pltpu.async_copy(mm_v, post_hbm.at[pl.ds((base + b) * M, M)], bsem)
        return 0

    def drain(b, _):
        pltpu.make_async_copy(mm_v, post_hbm.at[pl.ds((base + b) * M, M)],
                              bsem).wait()
        return 0

    lax.fori_loop(0, BPW, fire, 0)
    lax.fori_loop(0, BPW, drain, 0)
    pltpu.async_copy(rows_v, post_hbm.at[idx_v], ssem).wait()


_sc_writer = functools.partial(
    pl.kernel,
    out_type=jax.ShapeDtypeStruct((B * M, C), jnp.float32),
    mesh=plsc.VectorSubcoreMesh(core_axis_name="c", subcore_axis_name="s"),
    scratch_types=[
        pltpu.VMEM((M, C), jnp.float32),
        pltpu.VMEM((BPW, C), jnp.float32),
        pltpu.VMEM((BPW,), jnp.int32),
        pltpu.SemaphoreType.DMA,
        pltpu.SemaphoreType.DMA,
    ],
)(_sc_writer_body)


def kernel(input_encoded, memory_mean, memory_logvar):
    del memory_logvar

    z, kl, flat_idx, new_rows = pl.pallas_call(
        _compute_body,
        out_shape=[
            jax.ShapeDtypeStruct((B, C), jnp.float32),
            jax.ShapeDtypeStruct((B,), jnp.float32),
            jax.ShapeDtypeStruct((B,), jnp.int32),
            jax.ShapeDtypeStruct((B, C), jnp.float32),
        ],
    )(input_encoded, memory_mean)

    post_flat = _sc_writer(memory_mean, new_rows, flat_idx)
    posterior = post_flat.reshape(B, M, C)

    return z, posterior, kl


# speedup vs baseline: 1.2028x; 1.0697x over previous
import functools
import math

import jax
import jax.numpy as jnp
from jax import lax
from jax.experimental import pallas as pl
from jax.experimental.pallas import tpu as pltpu
from jax.experimental.pallas import tpu_sc as plsc

ALPHA = 0.1
B, M, C = 1024, 128, 384
NC, NS = 2, 16
NW = NC * NS
BPW = B // NW


def _compute_body(x_ref, mm_ref, z_ref, kl_ref, idx_ref, rows_ref):
    X = x_ref[...]
    MM = mm_ref[...]
    sims = jax.lax.dot_general(X, MM, (((1,), (1,)), ((), ())),
                               preferred_element_type=jnp.float32)
    closest = jnp.argmax(sims, axis=1)
    onehot = (jax.lax.broadcasted_iota(jnp.int32, (B, M), 1)
              == closest[:, None])
    oh_f = onehot.astype(jnp.float32)
    gathered = jax.lax.dot_general(oh_f, MM, (((1,), (0,)), ((), ())),
                                   preferred_element_type=jnp.float32)
    diff = X - gathered
    delta = ALPHA * diff
    xsq = jnp.sum(X * X, axis=1)
    s_at = jnp.sum(sims * oh_f, axis=1)
    corr = (1.0 - ALPHA) * s_at + ALPHA * xsq
    scores = jnp.where(onehot, corr[:, None], sims) * (1.0 / math.sqrt(C))
    smax = jnp.max(scores, axis=1, keepdims=True)
    e = jnp.exp(scores - smax)
    w = e / jnp.sum(e, axis=1, keepdims=True)
    z = jax.lax.dot_general(w, MM, (((1,), (0,)), ((), ())),
                            preferred_element_type=jnp.float32)
    w_at = jnp.sum(w * oh_f, axis=1)
    z = z + w_at[:, None] * delta
    z_ref[...] = z
    kl_ref[...] = 0.5 * (jnp.sum(diff * diff, axis=1)
                         + jnp.sum((z - X) ** 2, axis=1))
    idx_ref[...] = closest + M * jax.lax.broadcasted_iota(jnp.int32, (B,), 0)
    rows_ref[...] = gathered + delta


def _broadcast_body(mm_ref, post_ref):
    MM = mm_ref[...]
    n = post_ref.shape[0] // M
    post_ref[...] = jnp.broadcast_to(MM[None], (n, M, C)).reshape(n * M, C)


def _sc_scatter_body(rows_hbm, idx_hbm, post_ref, rows_v, idx_v, ssem):
    wid = lax.axis_index("s") * NC + lax.axis_index("c")
    base = wid * BPW
    pltpu.sync_copy(rows_hbm.at[pl.ds(base, BPW)], rows_v)
    pltpu.sync_copy(idx_hbm.at[pl.ds(base, BPW)], idx_v)
    pltpu.async_copy(rows_v, post_ref.at[idx_v], ssem).wait()


_sc_scatter = functools.partial(
    pl.kernel,
    out_type=(),
    mesh=plsc.VectorSubcoreMesh(core_axis_name="c", subcore_axis_name="s"),
    scratch_types=[
        pltpu.VMEM((BPW, C), jnp.float32),
        pltpu.VMEM((BPW,), jnp.int32),
        pltpu.SemaphoreType.DMA,
    ],
)(_sc_scatter_body)


def kernel(input_encoded, memory_mean, memory_logvar):
    del memory_logvar

    z, kl, flat_idx, new_rows = pl.pallas_call(
        _compute_body,
        out_shape=[
            jax.ShapeDtypeStruct((B, C), jnp.float32),
            jax.ShapeDtypeStruct((B,), jnp.float32),
            jax.ShapeDtypeStruct((B,), jnp.int32),
            jax.ShapeDtypeStruct((B, C), jnp.float32),
        ],
    )(input_encoded, memory_mean)

    BB = 32
    post_flat = pl.pallas_call(
        _broadcast_body,
        grid=(B // BB,),
        in_specs=[pl.BlockSpec((M, C), lambda i: (0, 0))],
        out_specs=pl.BlockSpec((BB * M, C), lambda i: (i, 0)),
        out_shape=jax.ShapeDtypeStruct((B * M, C), jnp.float32),
    )(memory_mean)

    post_ref = jax.new_ref(post_flat)
    _sc_scatter(new_rows, flat_idx, post_ref)
    posterior = post_ref[...].reshape(B, M, C)

    return z, posterior, kl


# speedup vs baseline: 1.2175x; 1.0123x over previous
import functools
import math

import jax
import jax.numpy as jnp
from jax import lax
from jax.experimental import pallas as pl
from jax.experimental.pallas import tpu as pltpu
from jax.experimental.pallas import tpu_sc as plsc

ALPHA = 0.1
B, M, C = 1024, 128, 384
NC, NS = 2, 16
NW = NC * NS
BPW = B // NW


def _fused_body(x_ref, mm_ref, post_ref, z_ref, kl_ref, idx_ref, rows_ref):
    MM = mm_ref[...]
    n = post_ref.shape[0] // M
    post_ref[...] = jnp.broadcast_to(MM[None], (n, M, C)).reshape(n * M, C)

    @pl.when(pl.program_id(0) == pl.num_programs(0) - 1)
    def _():
        _compute(x_ref[...], MM, z_ref, kl_ref, idx_ref, rows_ref)


def _compute(X, MM, z_ref, kl_ref, idx_ref, rows_ref):
    sims = jax.lax.dot_general(X, MM, (((1,), (1,)), ((), ())),
                               preferred_element_type=jnp.float32)
    closest = jnp.argmax(sims, axis=1)
    onehot = (jax.lax.broadcasted_iota(jnp.int32, (B, M), 1)
              == closest[:, None])
    oh_f = onehot.astype(jnp.float32)
    gathered = jax.lax.dot_general(oh_f, MM, (((1,), (0,)), ((), ())),
                                   preferred_element_type=jnp.float32)
    diff = X - gathered
    delta = ALPHA * diff
    xsq = jnp.sum(X * X, axis=1)
    s_at = jnp.sum(sims * oh_f, axis=1)
    corr = (1.0 - ALPHA) * s_at + ALPHA * xsq
    scores = jnp.where(onehot, corr[:, None], sims) * (1.0 / math.sqrt(C))
    smax = jnp.max(scores, axis=1, keepdims=True)
    e = jnp.exp(scores - smax)
    w = e / jnp.sum(e, axis=1, keepdims=True)
    z = jax.lax.dot_general(w, MM, (((1,), (0,)), ((), ())),
                            preferred_element_type=jnp.float32)
    w_at = jnp.sum(w * oh_f, axis=1)
    z = z + w_at[:, None] * delta
    z_ref[...] = z
    kl_ref[...] = 0.5 * (jnp.sum(diff * diff, axis=1)
                         + jnp.sum((z - X) ** 2, axis=1))
    idx_ref[...] = closest + M * jax.lax.broadcasted_iota(jnp.int32, (B,), 0)
    rows_ref[...] = gathered + delta


def _sc_scatter_body(rows_hbm, idx_hbm, post_ref, rows_v, idx_v, ssem):
    wid = lax.axis_index("s") * NC + lax.axis_index("c")
    base = wid * BPW
    pltpu.sync_copy(rows_hbm.at[pl.ds(base, BPW)], rows_v)
    pltpu.sync_copy(idx_hbm.at[pl.ds(base, BPW)], idx_v)
    pltpu.async_copy(rows_v, post_ref.at[idx_v], ssem).wait()


_sc_scatter = functools.partial(
    pl.kernel,
    out_type=(),
    mesh=plsc.VectorSubcoreMesh(core_axis_name="c", subcore_axis_name="s"),
    scratch_types=[
        pltpu.VMEM((BPW, C), jnp.float32),
        pltpu.VMEM((BPW,), jnp.int32),
        pltpu.SemaphoreType.DMA,
    ],
)(_sc_scatter_body)


def kernel(input_encoded, memory_mean, memory_logvar):
    del memory_logvar

    BB = 32
    post_flat, z, kl, flat_idx, new_rows = pl.pallas_call(
        _fused_body,
        grid=(B // BB,),
        in_specs=[
            pl.BlockSpec((B, C), lambda i: (0, 0)),
            pl.BlockSpec((M, C), lambda i: (0, 0)),
        ],
        out_specs=[
            pl.BlockSpec((BB * M, C), lambda i: (i, 0)),
            pl.BlockSpec((B, C), lambda i: (0, 0)),
            pl.BlockSpec((B,), lambda i: (0,)),
            pl.BlockSpec((B,), lambda i: (0,)),
            pl.BlockSpec((B, C), lambda i: (0, 0)),
        ],
        out_shape=[
            jax.ShapeDtypeStruct((B * M, C), jnp.float32),
            jax.ShapeDtypeStruct((B, C), jnp.float32),
            jax.ShapeDtypeStruct((B,), jnp.float32),
            jax.ShapeDtypeStruct((B,), jnp.int32),
            jax.ShapeDtypeStruct((B, C), jnp.float32),
        ],
    )(input_encoded, memory_mean)

    post_ref = jax.new_ref(post_flat)
    _sc_scatter(new_rows, flat_idx, post_ref)
    posterior = post_ref[...].reshape(B, M, C)

    return z, posterior, kl
